# baseline (speedup 1.0000x reference)
import functools

import jax
import jax.numpy as jnp
from jax import lax
from jax.experimental import pallas as pl
from jax.experimental.pallas import tpu as pltpu
from jax.experimental.pallas import tpu_sc as plsc

N = 10000
E = 320000
D = 128
H1 = 128
NP = 10240
NC, NS = 2, 16
NW = NC * NS
EPW = E // NW
C = 125
NCHUNK = EPW // C
RPT = NP // NS
OCH = 80
NB = 10
BR = N // NB


def _deg_body(src_hbm, out_hbm, onesv, sidx2, stagev, dego_sh, sem):
    c = lax.axis_index("c")
    s = lax.axis_index("s")
    wid = c * NS + s

    pltpu.sync_copy(src_hbm.at[pl.ds(wid * NCHUNK, NCHUNK)], sidx2)

    def _z(j, _):
        stagev[pl.ds(j * 16, 16)] = jnp.zeros((16,), jnp.float32)
        return 0
    lax.fori_loop(0, RPT // 16, _z, 0)
    pltpu.sync_copy(stagev, dego_sh.at[pl.ds(s * RPT, RPT)])

    def _o(j, _):
        onesv[pl.ds(j * 16, 16)] = jnp.ones((16,), jnp.float32)
        return 0
    lax.fori_loop(0, 8, _o, 0)
    plsc.subcore_barrier()

    ones = onesv.at[pl.ds(0, C)]

    def _fire(j, _):
        pltpu.async_copy(ones, dego_sh.at[sidx2.at[j]], sem, add=True)
        return 0
    lax.fori_loop(0, NCHUNK, _fire, 0)

    def _drain(j, _):
        pltpu.make_async_copy(ones, dego_sh.at[sidx2.at[0]], sem).wait()
        return 0
    lax.fori_loop(0, NCHUNK, _drain, 0)
    plsc.subcore_barrier()

    pltpu.sync_copy(dego_sh.at[pl.ds(s * RPT, RPT)], stagev)
    pltpu.sync_copy(stagev, out_hbm.at[c, s])


_deg_kernel = functools.partial(
    pl.kernel,
    out_type=jax.ShapeDtypeStruct((NC, NS, RPT), jnp.float32),
    mesh=plsc.VectorSubcoreMesh(core_axis_name="c", subcore_axis_name="s",
                                num_cores=NC, num_subcores=NS),
    scratch_types=[
        pltpu.VMEM((128,), jnp.float32),
        pltpu.VMEM((NCHUNK, C), jnp.int32),
        pltpu.VMEM((RPT,), jnp.float32),
        pltpu.VMEM_SHARED((NP,), jnp.float32),
        pltpu.SemaphoreType.DMA,
    ],
)(_deg_body)


def _scatter_body(g_hbm, src_hbm, dst_hbm, out_hbm, degi_hbm,
                  sidxa, sidxb, didx2, rows0, rows1, onesv, stagev,
                  acc_sh, degi_sh, sem0, sem1, semi0, semi1, sems0, sems1,
                  semo):
    c = lax.axis_index("c")
    s = lax.axis_index("s")
    wid = c * NS + s

    pltpu.sync_copy(dst_hbm.at[pl.ds(wid * NCHUNK, NCHUNK)], didx2)

    def _z(j, _):
        r = j // (D // 16)
        k = j % (D // 16)
        rows0[r, pl.ds(k * 16, 16)] = jnp.zeros((16,), jnp.float32)
        return 0
    lax.fori_loop(0, OCH * (D // 16), _z, 0)

    def _zc(r, _):
        pltpu.sync_copy(rows0.at[pl.ds(0, OCH)],
                        acc_sh.at[pl.ds(s * RPT + r * OCH, OCH)])
        return 0
    lax.fori_loop(0, RPT // OCH, _zc, 0)

    def _zd(j, _):
        stagev[pl.ds(j * 16, 16)] = jnp.zeros((16,), jnp.float32)
        return 0
    lax.fori_loop(0, RPT // 16, _zd, 0)
    pltpu.sync_copy(stagev, degi_sh.at[pl.ds(s * RPT, RPT)])

    def _o(j, _):
        onesv[pl.ds(j * 16, 16)] = jnp.ones((16,), jnp.float32)
        return 0
    lax.fori_loop(0, 8, _o, 0)
    plsc.subcore_barrier()

    ones = onesv.at[pl.ds(0, C)]

    def _idxload(j, buf, sem):
        return pltpu.make_async_copy(src_hbm.at[wid * NCHUNK + j], buf, sem)

    def _gather(buf, rows, sem):
        return pltpu.make_async_copy(g_hbm.at[buf], rows, sem)

    def _scat_start(rows, j, sem):
        pltpu.async_copy(rows, acc_sh.at[didx2.at[j]], sem, add=True)

    def _scat_wait(rows, sem):
        pltpu.make_async_copy(rows, acc_sh.at[didx2.at[0]], sem).wait()

    def _ones(j):
        pltpu.async_copy(ones, degi_sh.at[didx2.at[j]], semo, add=True)

    _idxload(0, sidxa, semi0).start()
    _idxload(1, sidxb, semi1).start()
    _idxload(0, sidxa, semi0).wait()
    _gather(sidxa, rows0, sem0).start()

    def _loop(i, _):
        j0 = 2 * i
        _gather(sidxa, rows0, sem0).wait()
        _scat_start(rows0, j0, sems0)
        _ones(j0)

        @pl.when(j0 + 2 < NCHUNK)
        def _():
            _idxload(j0 + 2, sidxa, semi0).start()

        _idxload(j0 + 1, sidxb, semi1).wait()

        @pl.when(i > 0)
        def _():
            _scat_wait(rows1, sems1)

        _gather(sidxb, rows1, sem1).start()
        _gather(sidxb, rows1, sem1).wait()
        _scat_start(rows1, j0 + 1, sems1)
        _ones(j0 + 1)

        @pl.when(j0 + 3 < NCHUNK)
        def _():
            _idxload(j0 + 3, sidxb, semi1).start()

        _scat_wait(rows0, sems0)

        @pl.when(j0 + 2 < NCHUNK)
        def _():
            _idxload(j0 + 2, sidxa, semi0).wait()
            _gather(sidxa, rows0, sem0).start()

        return 0
    lax.fori_loop(0, NCHUNK // 2, _loop, 0)
    _scat_wait(rows1, sems1)

    def _dr(j, _):
        pltpu.make_async_copy(ones, degi_sh.at[didx2.at[0]], semo).wait()
        return 0
    lax.fori_loop(0, NCHUNK, _dr, 0)
    plsc.subcore_barrier()

    def _out(r, _):
        row0 = s * RPT + r * OCH
        pltpu.sync_copy(acc_sh.at[pl.ds(row0, OCH)], rows0.at[pl.ds(0, OCH)])
        pltpu.sync_copy(rows0.at[pl.ds(0, OCH)], out_hbm.at[c, pl.ds(row0, OCH)])
        return 0
    lax.fori_loop(0, RPT // OCH, _out, 0)

    pltpu.sync_copy(degi_sh.at[pl.ds(s * RPT, RPT)], stagev)
    pltpu.sync_copy(stagev, degi_hbm.at[c, s])


_scatter_kernel = functools.partial(
    pl.kernel,
    out_type=[jax.ShapeDtypeStruct((NC, NP, D), jnp.float32),
              jax.ShapeDtypeStruct((NC, NS, RPT), jnp.float32)],
    mesh=plsc.VectorSubcoreMesh(core_axis_name="c", subcore_axis_name="s",
                                num_cores=NC, num_subcores=NS),
    scratch_types=[
        pltpu.VMEM((C,), jnp.int32),
        pltpu.VMEM((C,), jnp.int32),
        pltpu.VMEM((NCHUNK, C), jnp.int32),
        pltpu.VMEM((C, D), jnp.float32),
        pltpu.VMEM((C, D), jnp.float32),
        pltpu.VMEM((128,), jnp.float32),
        pltpu.VMEM((RPT,), jnp.float32),
        pltpu.VMEM_SHARED((NP, D), jnp.float32),
        pltpu.VMEM_SHARED((NP,), jnp.float32),
        pltpu.SemaphoreType.DMA,
        pltpu.SemaphoreType.DMA,
        pltpu.SemaphoreType.DMA,
        pltpu.SemaphoreType.DMA,
        pltpu.SemaphoreType.DMA,
        pltpu.SemaphoreType.DMA,
        pltpu.SemaphoreType.DMA,
    ],
)(_scatter_body)


def _g_body(deg_ref, x_ref, wc_ref, g_ref):
    deg_out = deg_ref[:, 0] + deg_ref[:, 1] + 1.0
    norm_src = lax.rsqrt(deg_out)
    h = jnp.dot(x_ref[...], wc_ref[...], preferred_element_type=jnp.float32)
    g_ref[...] = h * norm_src[:, None]


def _g_call(degT, x, Wc):
    return pl.pallas_call(
        _g_body,
        grid=(NB,),
        in_specs=[
            pl.BlockSpec((BR, NC), lambda i: (i, 0)),
            pl.BlockSpec((BR, D), lambda i: (i, 0)),
            pl.BlockSpec((D, H1), lambda i: (0, 0)),
        ],
        out_specs=pl.BlockSpec((BR, H1), lambda i: (i, 0)),
        out_shape=jax.ShapeDtypeStruct((N, H1), jnp.float32),
    )(degT, x, Wc)


def _fin_body(deg_ref, s_ref, g_ref, bc_ref, w1_ref, b1_ref, w2_ref, b2_ref,
              w3_ref, b3_ref, out_ref, acc):
    i = pl.program_id(0)
    deg_in = deg_ref[:, 0] + deg_ref[:, 1] + 1.0
    norm_dst = lax.rsqrt(deg_in)
    rows = s_ref[0] + s_ref[1] + g_ref[...]
    a = jnp.maximum(rows * norm_dst[:, None] + bc_ref[...][None, :], 0.0)
    bm = jnp.max(a, axis=0, keepdims=True)

    @pl.when(i == 0)
    def _():
        acc[...] = jnp.zeros_like(acc)

    acc[...] = jnp.maximum(acc[...], jnp.broadcast_to(bm, acc.shape))

    @pl.when(i == NB - 1)
    def _():
        hg = acc[0:1, :]
        a1 = jnp.maximum(
            jnp.dot(hg, w1_ref[...], preferred_element_type=jnp.float32)
            + b1_ref[...][None, :], 0.0)
        a2 = jnp.maximum(
            jnp.dot(a1, w2_ref[...], preferred_element_type=jnp.float32)
            + b2_ref[...][None, :], 0.0)
        out_ref[...] = (
            jnp.dot(a2, w3_ref[...], preferred_element_type=jnp.float32)
            + b3_ref[...][None, :])


def _fin_call(degT, S, g, bc, W1, b1, W2, b2, W3, b3):
    return pl.pallas_call(
        _fin_body,
        grid=(NB,),
        in_specs=[
            pl.BlockSpec((BR, NC), lambda i: (i, 0)),
            pl.BlockSpec((NC, BR, D), lambda i: (0, i, 0)),
            pl.BlockSpec((BR, H1), lambda i: (i, 0)),
            pl.BlockSpec((H1,), lambda i: (0,)),
            pl.BlockSpec((H1, 256), lambda i: (0, 0)),
            pl.BlockSpec((256,), lambda i: (0,)),
            pl.BlockSpec((256, 128), lambda i: (0, 0)),
            pl.BlockSpec((128,), lambda i: (0,)),
            pl.BlockSpec((128, 10), lambda i: (0, 0)),
            pl.BlockSpec((10,), lambda i: (0,)),
        ],
        out_specs=pl.BlockSpec((1, 10), lambda i: (0, 0)),
        out_shape=jax.ShapeDtypeStruct((1, 10), jnp.float32),
        scratch_shapes=[pltpu.VMEM((8, H1), jnp.float32)],
    )(degT, S, g, bc, W1, b1, W2, b2, W3, b3)


def kernel(x, edge_index, Wc, bc, W1, b1, W2, b2, W3, b3):
    src = edge_index[0]
    dst = edge_index[1]
    src3 = src.reshape(NW * NCHUNK, C)
    dst3 = dst.reshape(NW * NCHUNK, C)
    dego = _deg_kernel(src3)
    degoT = dego.reshape(NC, NP).transpose(1, 0)
    g = _g_call(degoT, x, Wc)
    S, degi = _scatter_kernel(g, src3, dst3)
    degiT = degi.reshape(NC, NP).transpose(1, 0)
    out = _fin_call(degiT, S, g, bc, W1, b1, W2, b2, W3, b3)
    return jnp.squeeze(out)


# speedup vs baseline: 1.1195x; 1.1195x over previous
import functools

import jax
import jax.numpy as jnp
from jax import lax
from jax.experimental import pallas as pl
from jax.experimental.pallas import tpu as pltpu
from jax.experimental.pallas import tpu_sc as plsc

N = 10000
E = 320000
D = 128
H1 = 128
NP = 10240
NC, NS = 2, 16
NW = NC * NS
EPW = E // NW
C = 125
NCHUNK = EPW // C
RPT = NP // NS
OCH = 80
NB = 10
BR = N // NB


def _deg_body(src_hbm, out_hbm, onesv, sidx2, stagev, dego_sh, sem):
    c = lax.axis_index("c")
    s = lax.axis_index("s")
    wid = c * NS + s

    pltpu.sync_copy(src_hbm.at[pl.ds(wid * NCHUNK, NCHUNK)], sidx2)

    def _z(j, _):
        stagev[pl.ds(j * 16, 16)] = jnp.zeros((16,), jnp.float32)
        return 0
    lax.fori_loop(0, RPT // 16, _z, 0)
    pltpu.sync_copy(stagev, dego_sh.at[pl.ds(s * RPT, RPT)])

    def _o(j, _):
        onesv[pl.ds(j * 16, 16)] = jnp.ones((16,), jnp.float32)
        return 0
    lax.fori_loop(0, 8, _o, 0)
    plsc.subcore_barrier()

    ones = onesv.at[pl.ds(0, C)]

    def _fire(j, _):
        pltpu.async_copy(ones, dego_sh.at[sidx2.at[j]], sem, add=True)
        return 0
    lax.fori_loop(0, NCHUNK, _fire, 0)

    def _drain(j, _):
        pltpu.make_async_copy(ones, dego_sh.at[sidx2.at[0]], sem).wait()
        return 0
    lax.fori_loop(0, NCHUNK, _drain, 0)
    plsc.subcore_barrier()

    pltpu.sync_copy(dego_sh.at[pl.ds(s * RPT, RPT)], stagev)
    pltpu.sync_copy(stagev, out_hbm.at[c, s])


_deg_kernel = functools.partial(
    pl.kernel,
    out_type=jax.ShapeDtypeStruct((NC, NS, RPT), jnp.float32),
    mesh=plsc.VectorSubcoreMesh(core_axis_name="c", subcore_axis_name="s",
                                num_cores=NC, num_subcores=NS),
    scratch_types=[
        pltpu.VMEM((128,), jnp.float32),
        pltpu.VMEM((NCHUNK, C), jnp.int32),
        pltpu.VMEM((RPT,), jnp.float32),
        pltpu.VMEM_SHARED((NP,), jnp.float32),
        pltpu.SemaphoreType.DMA,
    ],
)(_deg_body)


def _scatter_body(g_hbm, src_hbm, dst_hbm, out_hbm, degi_hbm,
                  sidxa, sidxb, didx2, rows0, rows1, onesv, stagev,
                  acc_sh, degi_sh, sem0, sem1, semi0, semi1, semo):
    c = lax.axis_index("c")
    s = lax.axis_index("s")
    wid = c * NS + s

    pltpu.sync_copy(dst_hbm.at[pl.ds(wid * NCHUNK, NCHUNK)], didx2)

    def _z(j, _):
        r = j // (D // 16)
        k = j % (D // 16)
        rows0[r, pl.ds(k * 16, 16)] = jnp.zeros((16,), jnp.float32)
        return 0
    lax.fori_loop(0, OCH * (D // 16), _z, 0)

    def _zc(r, _):
        pltpu.sync_copy(rows0.at[pl.ds(0, OCH)],
                        acc_sh.at[pl.ds(s * RPT + r * OCH, OCH)])
        return 0
    lax.fori_loop(0, RPT // OCH, _zc, 0)

    def _zd(j, _):
        stagev[pl.ds(j * 16, 16)] = jnp.zeros((16,), jnp.float32)
        return 0
    lax.fori_loop(0, RPT // 16, _zd, 0)
    pltpu.sync_copy(stagev, degi_sh.at[pl.ds(s * RPT, RPT)])

    def _o(j, _):
        onesv[pl.ds(j * 16, 16)] = jnp.ones((16,), jnp.float32)
        return 0
    lax.fori_loop(0, 8, _o, 0)
    plsc.subcore_barrier()

    ones = onesv.at[pl.ds(0, C)]

    def _idxload(j, buf, sem):
        return pltpu.make_async_copy(src_hbm.at[wid * NCHUNK + j], buf, sem)

    def _gather(buf, rows, sem):
        return pltpu.make_async_copy(g_hbm.at[buf], rows, sem)

    def _ones(j):
        pltpu.async_copy(ones, degi_sh.at[didx2.at[j]], semo, add=True)

    _idxload(0, sidxa, semi0).start()
    _idxload(1, sidxb, semi1).start()
    _idxload(0, sidxa, semi0).wait()
    _gather(sidxa, rows0, sem0).start()

    def _loop(i, _):
        j0 = 2 * i
        _idxload(j0 + 1, sidxb, semi1).wait()
        _gather(sidxb, rows1, sem1).start()
        _gather(sidxa, rows0, sem0).wait()

        @pl.when(j0 + 2 < NCHUNK)
        def _():
            _idxload(j0 + 2, sidxa, semi0).start()

        pltpu.sync_copy(rows0, acc_sh.at[didx2.at[j0]], add=True)
        _ones(j0)

        @pl.when(j0 + 2 < NCHUNK)
        def _():
            _idxload(j0 + 2, sidxa, semi0).wait()
            _gather(sidxa, rows0, sem0).start()

        _gather(sidxb, rows1, sem1).wait()

        @pl.when(j0 + 3 < NCHUNK)
        def _():
            _idxload(j0 + 3, sidxb, semi1).start()

        pltpu.sync_copy(rows1, acc_sh.at[didx2.at[j0 + 1]], add=True)
        _ones(j0 + 1)
        return 0
    lax.fori_loop(0, NCHUNK // 2, _loop, 0)

    def _dr(j, _):
        pltpu.make_async_copy(ones, degi_sh.at[didx2.at[0]], semo).wait()
        return 0
    lax.fori_loop(0, NCHUNK, _dr, 0)
    plsc.subcore_barrier()

    def _out(r, _):
        row0 = s * RPT + r * OCH
        pltpu.sync_copy(acc_sh.at[pl.ds(row0, OCH)], rows0.at[pl.ds(0, OCH)])
        pltpu.sync_copy(rows0.at[pl.ds(0, OCH)], out_hbm.at[c, pl.ds(row0, OCH)])
        return 0
    lax.fori_loop(0, RPT // OCH, _out, 0)

    pltpu.sync_copy(degi_sh.at[pl.ds(s * RPT, RPT)], stagev)
    pltpu.sync_copy(stagev, degi_hbm.at[c, s])


_scatter_kernel = functools.partial(
    pl.kernel,
    out_type=[jax.ShapeDtypeStruct((NC, NP, D), jnp.float32),
              jax.ShapeDtypeStruct((NC, NS, RPT), jnp.float32)],
    mesh=plsc.VectorSubcoreMesh(core_axis_name="c", subcore_axis_name="s",
                                num_cores=NC, num_subcores=NS),
    scratch_types=[
        pltpu.VMEM((C,), jnp.int32),
        pltpu.VMEM((C,), jnp.int32),
        pltpu.VMEM((NCHUNK, C), jnp.int32),
        pltpu.VMEM((C, D), jnp.float32),
        pltpu.VMEM((C, D), jnp.float32),
        pltpu.VMEM((128,), jnp.float32),
        pltpu.VMEM((RPT,), jnp.float32),
        pltpu.VMEM_SHARED((NP, D), jnp.float32),
        pltpu.VMEM_SHARED((NP,), jnp.float32),
        pltpu.SemaphoreType.DMA,
        pltpu.SemaphoreType.DMA,
        pltpu.SemaphoreType.DMA,
        pltpu.SemaphoreType.DMA,
        pltpu.SemaphoreType.DMA,
    ],
)(_scatter_body)


def _g_body(deg_ref, x_ref, wc_ref, g_ref):
    deg_out = deg_ref[:, 0] + deg_ref[:, 1] + 1.0
    norm_src = lax.rsqrt(deg_out)
    h = jnp.dot(x_ref[...], wc_ref[...], preferred_element_type=jnp.float32)
    g_ref[...] = h * norm_src[:, None]


def _g_call(degT, x, Wc):
    return pl.pallas_call(
        _g_body,
        grid=(NB,),
        in_specs=[
            pl.BlockSpec((BR, NC), lambda i: (i, 0)),
            pl.BlockSpec((BR, D), lambda i: (i, 0)),
            pl.BlockSpec((D, H1), lambda i: (0, 0)),
        ],
        out_specs=pl.BlockSpec((BR, H1), lambda i: (i, 0)),
        out_shape=jax.ShapeDtypeStruct((N, H1), jnp.float32),
    )(degT, x, Wc)


def _fin_body(deg_ref, s_ref, g_ref, bc_ref, w1_ref, b1_ref, w2_ref, b2_ref,
              w3_ref, b3_ref, out_ref, acc):
    i = pl.program_id(0)
    deg_in = deg_ref[:, 0] + deg_ref[:, 1] + 1.0
    norm_dst = lax.rsqrt(deg_in)
    rows = s_ref[0] + s_ref[1] + g_ref[...]
    a = jnp.maximum(rows * norm_dst[:, None] + bc_ref[...][None, :], 0.0)
    bm = jnp.max(a, axis=0, keepdims=True)

    @pl.when(i == 0)
    def _():
        acc[...] = jnp.zeros_like(acc)

    acc[...] = jnp.maximum(acc[...], jnp.broadcast_to(bm, acc.shape))

    @pl.when(i == NB - 1)
    def _():
        hg = acc[0:1, :]
        a1 = jnp.maximum(
            jnp.dot(hg, w1_ref[...], preferred_element_type=jnp.float32)
            + b1_ref[...][None, :], 0.0)
        a2 = jnp.maximum(
            jnp.dot(a1, w2_ref[...], preferred_element_type=jnp.float32)
            + b2_ref[...][None, :], 0.0)
        out_ref[...] = (
            jnp.dot(a2, w3_ref[...], preferred_element_type=jnp.float32)
            + b3_ref[...][None, :])


def _fin_call(degT, S, g, bc, W1, b1, W2, b2, W3, b3):
    return pl.pallas_call(
        _fin_body,
        grid=(NB,),
        in_specs=[
            pl.BlockSpec((BR, NC), lambda i: (i, 0)),
            pl.BlockSpec((NC, BR, D), lambda i: (0, i, 0)),
            pl.BlockSpec((BR, H1), lambda i: (i, 0)),
            pl.BlockSpec((H1,), lambda i: (0,)),
            pl.BlockSpec((H1, 256), lambda i: (0, 0)),
            pl.BlockSpec((256,), lambda i: (0,)),
            pl.BlockSpec((256, 128), lambda i: (0, 0)),
            pl.BlockSpec((128,), lambda i: (0,)),
            pl.BlockSpec((128, 10), lambda i: (0, 0)),
            pl.BlockSpec((10,), lambda i: (0,)),
        ],
        out_specs=pl.BlockSpec((1, 10), lambda i: (0, 0)),
        out_shape=jax.ShapeDtypeStruct((1, 10), jnp.float32),
        scratch_shapes=[pltpu.VMEM((8, H1), jnp.float32)],
    )(degT, S, g, bc, W1, b1, W2, b2, W3, b3)


def kernel(x, edge_index, Wc, bc, W1, b1, W2, b2, W3, b3):
    src = edge_index[0]
    dst = edge_index[1]
    src3 = src.reshape(NW * NCHUNK, C)
    dst3 = dst.reshape(NW * NCHUNK, C)
    dego = _deg_kernel(src3)
    degoT = dego.reshape(NC, NP).transpose(1, 0)
    g = _g_call(degoT, x, Wc)
    S, degi = _scatter_kernel(g, src3, dst3)
    degiT = degi.reshape(NC, NP).transpose(1, 0)
    out = _fin_call(degiT, S, g, bc, W1, b1, W2, b2, W3, b3)
    return jnp.squeeze(out)
